# Initial kernel scaffold; baseline (speedup 1.0000x reference)
import jax
import jax.numpy as jnp
from jax.experimental import pallas as pl


def kernel(distance_matrix, target):
    raise NotImplementedError("write your pallas kernel here")


# speedup vs baseline: 1.1701x; 1.1701x over previous
import functools

import jax
import jax.numpy as jnp
from jax.experimental import pallas as pl
from jax.experimental.pallas import tpu as pltpu


def _cdiv(a, b):
    return (a + b - 1) // b


def _sort_topk_body(d_ref, t_ref, gt_ref, ki_ref, kv_ref, *, n, kpad):
    b = d_ref.shape[0]
    v = d_ref[...]
    t = t_ref[...]
    gt_ref[...] = jnp.broadcast_to(jnp.sum(v * t), (1, 1, 1))

    idx = jax.lax.broadcasted_iota(jnp.int32, (b, n), 1)
    col = jax.lax.broadcasted_iota(jnp.int32, (1, n), 1)

    size = 2
    while size <= n:
        asc = (col & size) == 0
        j = size // 2
        while j >= 1:
            low = (col & j) == 0
            vp = jnp.where(low, pltpu.roll(v, n - j, 1), pltpu.roll(v, j, 1))
            ip = jnp.where(low, pltpu.roll(idx, n - j, 1), pltpu.roll(idx, j, 1))
            less = (v < vp) | ((v == vp) & (idx < ip))
            take_min = low == asc
            sel = less == take_min
            v = jnp.where(sel, v, vp)
            idx = jnp.where(sel, idx, ip)
            j //= 2
        size *= 2

    ki_ref[...] = idx[:, :kpad]
    kv_ref[...] = v[:, :kpad]


def _topk_call(distance_matrix, target):
    r, n = distance_matrix.shape
    k = min(r, 10 + 2 * (r // 10))
    kpad = min(_cdiv(k, 128) * 128, n)
    b = min(128, r)
    g = r // b
    gt_p, ki, kv = pl.pallas_call(
        functools.partial(_sort_topk_body, n=n, kpad=kpad),
        grid=(g,),
        in_specs=[
            pl.BlockSpec((b, n), lambda i: (i, 0)),
            pl.BlockSpec((b, n), lambda i: (i, 0)),
        ],
        out_specs=[
            pl.BlockSpec((1, 1, 1), lambda i: (i, 0, 0)),
            pl.BlockSpec((b, kpad), lambda i: (i, 0)),
            pl.BlockSpec((b, kpad), lambda i: (i, 0)),
        ],
        out_shape=[
            jax.ShapeDtypeStruct((g, 1, 1), jnp.float32),
            jax.ShapeDtypeStruct((r, kpad), jnp.int32),
            jax.ShapeDtypeStruct((r, kpad), jnp.float32),
        ],
        compiler_params=pltpu.CompilerParams(
            dimension_semantics=("parallel",)
        ),
    )(distance_matrix, target)
    return gt_p, ki[:, :k], kv[:, :k]


def kernel(distance_matrix, target):
    r, n = distance_matrix.shape
    k = min(r, 10 + 2 * (r // 10))

    gt_p, ki, kv = _topk_call(distance_matrix, target)
    gt = jnp.sum(gt_p)

    rows = jax.lax.broadcasted_iota(jnp.int32, (r, k), 0)
    dst = ki + r
    e0 = jnp.stack([rows, dst], axis=2).reshape(-1)
    e1 = jnp.stack([dst, rows], axis=2).reshape(-1)
    edge_index = jnp.stack([e0, e1], axis=0)
    edge_attr = jnp.stack([kv, kv], axis=2).reshape(-1, 1)

    x = jnp.zeros((r + n, 8), dtype=jnp.float32)
    y = target.reshape(-1, 1)
    cost_vec = distance_matrix.reshape(-1, 1)
    return (gt, x, edge_index, edge_attr, y, cost_vec)


# speedup vs baseline: 1.3256x; 1.1329x over previous
import functools

import jax
import jax.numpy as jnp
from jax.experimental import pallas as pl
from jax.experimental.pallas import tpu as pltpu


def _cdiv(a, b):
    return (a + b - 1) // b


def _sort_topk_body(d_ref, t_ref, gt_ref, ki_ref, kv_ref, *, n, kpad):
    bcols = d_ref.shape[1]
    v = d_ref[...]
    t = t_ref[...]
    gt_ref[...] = jnp.broadcast_to(jnp.sum(v * t), (1, 1, 1))

    idx = jax.lax.broadcasted_iota(jnp.int32, (n, bcols), 0)
    pos = jax.lax.broadcasted_iota(jnp.int32, (n, 1), 0)

    size = 2
    while size <= n:
        asc = (pos & size) == 0
        j = size // 2
        while j >= 1:
            low = (pos & j) == 0
            vp = jnp.where(low, pltpu.roll(v, n - j, 0), pltpu.roll(v, j, 0))
            ip = jnp.where(low, pltpu.roll(idx, n - j, 0), pltpu.roll(idx, j, 0))
            less = (v < vp) | ((v == vp) & (idx < ip))
            take_min = low == asc
            sel = less == take_min
            v = jnp.where(sel, v, vp)
            idx = jnp.where(sel, idx, ip)
            j //= 2
        size *= 2

    ki_ref[...] = idx[:kpad, :]
    kv_ref[...] = v[:kpad, :]


def _topk_call(d_t, t_t):
    n, r = d_t.shape
    k = min(r, 10 + 2 * (r // 10))
    kpad = min(_cdiv(k, 8) * 8, n)
    b = min(128, r)
    g = r // b
    gt_p, ki, kv = pl.pallas_call(
        functools.partial(_sort_topk_body, n=n, kpad=kpad),
        grid=(g,),
        in_specs=[
            pl.BlockSpec((n, b), lambda i: (0, i)),
            pl.BlockSpec((n, b), lambda i: (0, i)),
        ],
        out_specs=[
            pl.BlockSpec((1, 1, 1), lambda i: (i, 0, 0)),
            pl.BlockSpec((kpad, b), lambda i: (0, i)),
            pl.BlockSpec((kpad, b), lambda i: (0, i)),
        ],
        out_shape=[
            jax.ShapeDtypeStruct((g, 1, 1), jnp.float32),
            jax.ShapeDtypeStruct((kpad, r), jnp.int32),
            jax.ShapeDtypeStruct((kpad, r), jnp.float32),
        ],
        compiler_params=pltpu.CompilerParams(
            dimension_semantics=("parallel",)
        ),
    )(d_t, t_t)
    return gt_p, ki, kv


def kernel(distance_matrix, target):
    r, n = distance_matrix.shape
    k = min(r, 10 + 2 * (r // 10))

    gt_p, ki_t, kv_t = _topk_call(distance_matrix.T, target.T)
    gt = jnp.sum(gt_p)
    ki = ki_t.T[:, :k]
    kv = kv_t.T[:, :k]

    rows = jax.lax.broadcasted_iota(jnp.int32, (r, k), 0)
    dst = ki + r
    e0 = jnp.stack([rows, dst], axis=2).reshape(-1)
    e1 = jnp.stack([dst, rows], axis=2).reshape(-1)
    edge_index = jnp.stack([e0, e1], axis=0)
    edge_attr = jnp.stack([kv, kv], axis=2).reshape(-1, 1)

    x = jnp.zeros((r + n, 8), dtype=jnp.float32)
    y = target.reshape(-1, 1)
    cost_vec = distance_matrix.reshape(-1, 1)
    return (gt, x, edge_index, edge_attr, y, cost_vec)


# speedup vs baseline: 1.4514x; 1.0949x over previous
import functools

import jax
import jax.numpy as jnp
from jax import lax
from jax.experimental import pallas as pl
from jax.experimental.pallas import tpu as pltpu

_CHUNK = 64


def _cdiv(a, b):
    return (a + b - 1) // b


def _cmp_exchange(v, idx, vp, ip, low, asc):
    less = (v < vp) | ((v == vp) & (idx < ip))
    sel = less == (low == asc)
    return jnp.where(sel, v, vp), jnp.where(sel, idx, ip)


def _stage(v, idx, pos, j, asc, m):
    low = (pos & j) == 0
    vp = jnp.where(low, pltpu.roll(v, m - j, 0), pltpu.roll(v, j, 0))
    ip = jnp.where(low, pltpu.roll(idx, m - j, 0), pltpu.roll(idx, j, 0))
    return _cmp_exchange(v, idx, vp, ip, low, asc)


def _sort_topk_body(d_ref, t_ref, gt_ref, ki_ref, kv_ref, vs_ref, is_ref,
                    *, n, kpad):
    b = d_ref.shape[1]
    c = min(_CHUNK, n)
    nch = n // c
    pos_c = lax.broadcasted_iota(jnp.int32, (c, 1), 0)

    def pass0(ci, acc):
        base = ci * c
        v = d_ref[pl.ds(base, c), :]
        acc = acc + jnp.sum(v * t_ref[pl.ds(base, c), :])
        idx = lax.broadcasted_iota(jnp.int32, (c, b), 0) + base
        pos = pos_c + base
        size = 2
        while size <= c:
            asc = (pos & size) == 0
            j = size // 2
            while j >= 1:
                v, idx = _stage(v, idx, pos_c, j, asc, c)
                j //= 2
            size *= 2
        vs_ref[pl.ds(base, c), :] = v
        is_ref[pl.ds(base, c), :] = idx
        return acc

    acc = lax.fori_loop(0, nch, pass0, jnp.float32(0.0))
    gt_ref[...] = jnp.broadcast_to(acc, (1, 1, 1))

    pos_f = lax.broadcasted_iota(jnp.int32, (n, 1), 0)
    size = 2 * c
    while size <= n:
        j = size // 2
        while j >= c:
            asc = (pos_f & size) == 0
            v = vs_ref[...]
            idx = is_ref[...]
            v, idx = _stage(v, idx, pos_f, j, asc, n)
            vs_ref[...] = v
            is_ref[...] = idx
            j //= 2

        def passf(ci, _, size=size):
            base = ci * c
            v = vs_ref[pl.ds(base, c), :]
            idx = is_ref[pl.ds(base, c), :]
            asc = ((pos_c + base) & size) == 0
            j = c // 2
            while j >= 1:
                v, idx = _stage(v, idx, pos_c, j, asc, c)
                j //= 2
            vs_ref[pl.ds(base, c), :] = v
            is_ref[pl.ds(base, c), :] = idx
            return 0

        lax.fori_loop(0, nch, passf, 0)
        size *= 2

    ki_ref[...] = is_ref[pl.ds(0, kpad), :]
    kv_ref[...] = vs_ref[pl.ds(0, kpad), :]


def _topk_call(d_t, t_t):
    n, r = d_t.shape
    k = min(r, 10 + 2 * (r // 10))
    kpad = min(_cdiv(k, 8) * 8, n)
    b = min(128, r)
    g = r // b
    gt_p, ki, kv = pl.pallas_call(
        functools.partial(_sort_topk_body, n=n, kpad=kpad),
        grid=(g,),
        in_specs=[
            pl.BlockSpec((n, b), lambda i: (0, i)),
            pl.BlockSpec((n, b), lambda i: (0, i)),
        ],
        out_specs=[
            pl.BlockSpec((1, 1, 1), lambda i: (i, 0, 0)),
            pl.BlockSpec((kpad, b), lambda i: (0, i)),
            pl.BlockSpec((kpad, b), lambda i: (0, i)),
        ],
        out_shape=[
            jax.ShapeDtypeStruct((g, 1, 1), jnp.float32),
            jax.ShapeDtypeStruct((kpad, r), jnp.int32),
            jax.ShapeDtypeStruct((kpad, r), jnp.float32),
        ],
        scratch_shapes=[
            pltpu.VMEM((n, b), jnp.float32),
            pltpu.VMEM((n, b), jnp.int32),
        ],
        compiler_params=pltpu.CompilerParams(
            dimension_semantics=("parallel",)
        ),
    )(d_t, t_t)
    return gt_p, ki, kv


def kernel(distance_matrix, target):
    r, n = distance_matrix.shape
    k = min(r, 10 + 2 * (r // 10))

    gt_p, ki_t, kv_t = _topk_call(distance_matrix.T, target.T)
    gt = jnp.sum(gt_p)
    ki = ki_t.T[:, :k]
    kv = kv_t.T[:, :k]

    rows = lax.broadcasted_iota(jnp.int32, (r, k), 0)
    dst = ki + r
    e0 = jnp.stack([rows, dst], axis=2).reshape(-1)
    e1 = jnp.stack([dst, rows], axis=2).reshape(-1)
    edge_index = jnp.stack([e0, e1], axis=0)
    edge_attr = jnp.stack([kv, kv], axis=2).reshape(-1, 1)

    x = jnp.zeros((r + n, 8), dtype=jnp.float32)
    y = target.reshape(-1, 1)
    cost_vec = distance_matrix.reshape(-1, 1)
    return (gt, x, edge_index, edge_attr, y, cost_vec)


# speedup vs baseline: 1.4614x; 1.0068x over previous
import functools

import jax
import jax.numpy as jnp
from jax import lax
from jax.experimental import pallas as pl
from jax.experimental.pallas import tpu as pltpu

_CHUNK = 64


def _cdiv(a, b):
    return (a + b - 1) // b


def _cmp_exchange(v, idx, vp, ip, low, asc):
    less = (v < vp) | ((v == vp) & (idx < ip))
    sel = less == (low == asc)
    return jnp.where(sel, v, vp), jnp.where(sel, idx, ip)


def _stage(v, idx, pos, j, asc, m):
    low = (pos & j) == 0
    vp = jnp.where(low, pltpu.roll(v, m - j, 0), pltpu.roll(v, j, 0))
    ip = jnp.where(low, pltpu.roll(idx, m - j, 0), pltpu.roll(idx, j, 0))
    return _cmp_exchange(v, idx, vp, ip, low, asc)


def _sort_topk_body(d_ref, t_ref, gt_ref, ki_ref, kv_ref, vs_ref, is_ref,
                    *, n, kpad):
    b = d_ref.shape[0]
    c = min(_CHUNK, n)
    nch = n // c
    pos_c = lax.broadcasted_iota(jnp.int32, (c, 1), 0)

    d0 = d_ref[...]
    gt_ref[...] = jnp.broadcast_to(jnp.sum(d0 * t_ref[...]), (1, 1, 1))
    vs_ref[...] = d0.T

    def pass0(ci, _):
        base = ci * c
        v = vs_ref[pl.ds(base, c), :]
        idx = lax.broadcasted_iota(jnp.int32, (c, b), 0) + base
        pos = pos_c + base
        size = 2
        while size <= c:
            asc = (pos & size) == 0
            j = size // 2
            while j >= 1:
                v, idx = _stage(v, idx, pos_c, j, asc, c)
                j //= 2
            size *= 2
        vs_ref[pl.ds(base, c), :] = v
        is_ref[pl.ds(base, c), :] = idx
        return 0

    lax.fori_loop(0, nch, pass0, 0)

    pos_f = lax.broadcasted_iota(jnp.int32, (n, 1), 0)
    size = 2 * c
    while size <= n:
        j = size // 2
        while j >= c:
            asc = (pos_f & size) == 0
            v = vs_ref[...]
            idx = is_ref[...]
            v, idx = _stage(v, idx, pos_f, j, asc, n)
            vs_ref[...] = v
            is_ref[...] = idx
            j //= 2

        def passf(ci, _, size=size):
            base = ci * c
            v = vs_ref[pl.ds(base, c), :]
            idx = is_ref[pl.ds(base, c), :]
            asc = ((pos_c + base) & size) == 0
            j = c // 2
            while j >= 1:
                v, idx = _stage(v, idx, pos_c, j, asc, c)
                j //= 2
            vs_ref[pl.ds(base, c), :] = v
            is_ref[pl.ds(base, c), :] = idx
            return 0

        lax.fori_loop(0, nch, passf, 0)
        size *= 2

    ki_ref[...] = is_ref[pl.ds(0, kpad), :].T
    kv_ref[...] = vs_ref[pl.ds(0, kpad), :].T


def _topk_call(d, t):
    r, n = d.shape
    k = min(r, 10 + 2 * (r // 10))
    kpad = min(_cdiv(k, 8) * 8, n)
    b = min(128, r)
    g = r // b
    gt_p, ki, kv = pl.pallas_call(
        functools.partial(_sort_topk_body, n=n, kpad=kpad),
        grid=(g,),
        in_specs=[
            pl.BlockSpec((b, n), lambda i: (i, 0)),
            pl.BlockSpec((b, n), lambda i: (i, 0)),
        ],
        out_specs=[
            pl.BlockSpec((1, 1, 1), lambda i: (i, 0, 0)),
            pl.BlockSpec((b, kpad), lambda i: (i, 0)),
            pl.BlockSpec((b, kpad), lambda i: (i, 0)),
        ],
        out_shape=[
            jax.ShapeDtypeStruct((g, 1, 1), jnp.float32),
            jax.ShapeDtypeStruct((r, kpad), jnp.int32),
            jax.ShapeDtypeStruct((r, kpad), jnp.float32),
        ],
        scratch_shapes=[
            pltpu.VMEM((n, b), jnp.float32),
            pltpu.VMEM((n, b), jnp.int32),
        ],
        compiler_params=pltpu.CompilerParams(
            dimension_semantics=("parallel",)
        ),
    )(d, t)
    return gt_p, ki, kv


def kernel(distance_matrix, target):
    r, n = distance_matrix.shape
    k = min(r, 10 + 2 * (r // 10))

    gt_p, ki, kv = _topk_call(distance_matrix, target)
    gt = jnp.sum(gt_p)
    ki = ki[:, :k]
    kv = kv[:, :k]

    rows = lax.broadcasted_iota(jnp.int32, (r, k), 0)
    dst = ki + r
    e0 = jnp.stack([rows, dst], axis=2).reshape(-1)
    e1 = jnp.stack([dst, rows], axis=2).reshape(-1)
    edge_index = jnp.stack([e0, e1], axis=0)
    edge_attr = jnp.stack([kv, kv], axis=2).reshape(-1, 1)

    x = jnp.zeros((r + n, 8), dtype=jnp.float32)
    y = target.reshape(-1, 1)
    cost_vec = distance_matrix.reshape(-1, 1)
    return (gt, x, edge_index, edge_attr, y, cost_vec)


# speedup vs baseline: 1.4617x; 1.0002x over previous
import functools

import jax
import jax.numpy as jnp
from jax import lax
from jax.experimental import pallas as pl
from jax.experimental.pallas import tpu as pltpu

_CHUNK = 64


def _cdiv(a, b):
    return (a + b - 1) // b


def _cmp_exchange(v, idx, vp, ip, low, asc):
    less = (v < vp) | ((v == vp) & (idx < ip))
    sel = less == (low == asc)
    return jnp.where(sel, v, vp), jnp.where(sel, idx, ip)


def _stage(v, idx, pos, j, asc, m):
    low = (pos & j) == 0
    vp = jnp.where(low, pltpu.roll(v, m - j, 0), pltpu.roll(v, j, 0))
    ip = jnp.where(low, pltpu.roll(idx, m - j, 0), pltpu.roll(idx, j, 0))
    return _cmp_exchange(v, idx, vp, ip, low, asc)


def _sort_topk_body(d_ref, t_ref, gt_ref, ki_ref, kv_ref, vs_ref, is_ref,
                    *, n, kpad):
    b = d_ref.shape[0]
    c = min(_CHUNK, n)
    nch = n // c
    pos_c = lax.broadcasted_iota(jnp.int32, (c, 1), 0)

    d0 = d_ref[...]
    gt_ref[...] = jnp.broadcast_to(jnp.sum(d0 * t_ref[...]), (1, 1, 1))
    vs_ref[...] = d0.T

    def pass0(ci, _):
        base = ci * c
        v = vs_ref[pl.ds(base, c), :]
        idx = lax.broadcasted_iota(jnp.int32, (c, b), 0) + base
        pos = pos_c + base
        size = 2
        while size <= c:
            asc = (pos & size) == 0
            j = size // 2
            while j >= 1:
                v, idx = _stage(v, idx, pos_c, j, asc, c)
                j //= 2
            size *= 2
        vs_ref[pl.ds(base, c), :] = v
        is_ref[pl.ds(base, c), :] = idx
        return 0

    lax.fori_loop(0, nch, pass0, 0)

    pos_f = lax.broadcasted_iota(jnp.int32, (n, 1), 0)
    size = 2 * c
    while size <= n:
        j = size // 2
        while j >= c:
            asc = (pos_f & size) == 0
            v = vs_ref[...]
            idx = is_ref[...]
            v, idx = _stage(v, idx, pos_f, j, asc, n)
            vs_ref[...] = v
            is_ref[...] = idx
            j //= 2

        def passf(ci, _, size=size):
            base = ci * c
            v = vs_ref[pl.ds(base, c), :]
            idx = is_ref[pl.ds(base, c), :]
            asc = ((pos_c + base) & size) == 0
            j = c // 2
            while j >= 1:
                v, idx = _stage(v, idx, pos_c, j, asc, c)
                j //= 2
            vs_ref[pl.ds(base, c), :] = v
            is_ref[pl.ds(base, c), :] = idx
            return 0

        lax.fori_loop(0, nch, passf, 0)
        size *= 2

    ki_ref[...] = is_ref[pl.ds(0, kpad), :].T
    kv_ref[...] = vs_ref[pl.ds(0, kpad), :].T


def _topk_call(d, t):
    r, n = d.shape
    k = min(r, 10 + 2 * (r // 10))
    kpad = min(_cdiv(k, 8) * 8, n)
    b = min(128, r)
    g = r // b
    gt_p, ki, kv = pl.pallas_call(
        functools.partial(_sort_topk_body, n=n, kpad=kpad),
        grid=(g,),
        in_specs=[
            pl.BlockSpec((b, n), lambda i: (i, 0)),
            pl.BlockSpec((b, n), lambda i: (i, 0)),
        ],
        out_specs=[
            pl.BlockSpec((1, 1, 1), lambda i: (i, 0, 0)),
            pl.BlockSpec((b, kpad), lambda i: (i, 0)),
            pl.BlockSpec((b, kpad), lambda i: (i, 0)),
        ],
        out_shape=[
            jax.ShapeDtypeStruct((g, 1, 1), jnp.float32),
            jax.ShapeDtypeStruct((r, kpad), jnp.int32),
            jax.ShapeDtypeStruct((r, kpad), jnp.float32),
        ],
        scratch_shapes=[
            pltpu.VMEM((n, b), jnp.float32),
            pltpu.VMEM((n, b), jnp.int32),
        ],
        compiler_params=pltpu.CompilerParams(
            dimension_semantics=("parallel",)
        ),
    )(d, t)
    return gt_p, ki, kv


def kernel(distance_matrix, target):
    r, n = distance_matrix.shape
    k = min(r, 10 + 2 * (r // 10))

    gt_p, ki, kv = _topk_call(distance_matrix, target)
    gt = jnp.sum(gt_p)
    ki = ki[:, :k]
    kv = kv[:, :k]

    rows = lax.broadcasted_iota(jnp.int32, (r, k), 0)
    dst = ki + r
    e0 = jnp.stack([rows, dst], axis=2).reshape(-1)
    e1 = jnp.stack([dst, rows], axis=2).reshape(-1)
    edge_index = jnp.stack([e0, e1], axis=0)
    edge_attr = jnp.stack([kv, kv], axis=2).reshape(-1, 1)

    x = jnp.zeros((r + n, 8), dtype=jnp.float32)
    y = target.reshape(-1)[:, None]
    cost_vec = distance_matrix.reshape(-1)[:, None]
    return (gt, x, edge_index, edge_attr, y, cost_vec)
